# Initial kernel scaffold; baseline (speedup 1.0000x reference)
import jax
import jax.numpy as jnp
from jax.experimental import pallas as pl


def kernel(dino_features, W1, b1, W2, b2):
    raise NotImplementedError("write your pallas kernel here")


# speedup vs baseline: 1.5505x; 1.5505x over previous
import jax
import jax.numpy as jnp
from jax.experimental import pallas as pl

B, H, W, C = 16, 32, 32, 384
HID = 128
HW = H * W


def _fused_kernel(x_ref, w1_ref, b1_ref, w2_ref, b2_ref, o_ref):
    x = x_ref[0].reshape(HW, C).astype(jnp.bfloat16)
    col = jax.lax.broadcasted_iota(jnp.int32, (HW, 1), 0) % W
    xl = jnp.where(col == 0, jnp.bfloat16(0), jnp.roll(x, 1, axis=0))
    xr = jnp.where(col == W - 1, jnp.bfloat16(0), jnp.roll(x, -1, axis=0))
    zrow = jnp.zeros((W, C), jnp.bfloat16)
    bufs = [
        jnp.concatenate([zrow, xl, zrow], axis=0),
        jnp.concatenate([zrow, x, zrow], axis=0),
        jnp.concatenate([zrow, xr, zrow], axis=0),
    ]
    acc = jnp.zeros((HW, HID), jnp.float32)
    for ky in range(3):
        for kx in range(3):
            tap = bufs[kx][W * ky:W * ky + HW]
            acc = acc + jnp.dot(tap, w1_ref[3 * ky + kx],
                                preferred_element_type=jnp.float32)
    h = jnp.maximum(acc + b1_ref[0][None, :], 0.0).astype(jnp.bfloat16)
    w2 = w2_ref[...].astype(jnp.bfloat16)
    logits = jax.lax.dot_general(w2, h, (((1,), (1,)), ((), ())),
                                 preferred_element_type=jnp.float32)
    o_ref[0] = jax.nn.sigmoid(logits + b2_ref[0, 0])


@jax.jit
def kernel(dino_features, W1, b1, W2, b2):
    w1r = jnp.transpose(W1, (2, 3, 1, 0)).reshape(9, C, HID).astype(jnp.bfloat16)
    out = pl.pallas_call(
        _fused_kernel,
        grid=(B,),
        in_specs=[
            pl.BlockSpec((1, H, W, C), lambda b: (b, 0, 0, 0)),
            pl.BlockSpec((9, C, HID), lambda b: (0, 0, 0)),
            pl.BlockSpec((1, HID), lambda b: (0, 0)),
            pl.BlockSpec((1, HID), lambda b: (0, 0)),
            pl.BlockSpec((1, 1), lambda b: (0, 0)),
        ],
        out_specs=pl.BlockSpec((1, 1, HW), lambda b: (b, 0, 0)),
        out_shape=jax.ShapeDtypeStruct((B, 1, HW), jnp.float32),
    )(dino_features, w1r, b1.reshape(1, HID), W2.reshape(1, HID),
      b2.reshape(1, 1))
    return out.reshape(B, H, W, 1)
